# baseline (speedup 1.0000x reference)
import functools

import jax
import jax.numpy as jnp
from jax import lax
from jax.experimental import pallas as pl
from jax.experimental.pallas import tpu as pltpu
from jax.experimental.pallas import tpu_sc as plsc

_NC = 2
_NS = 16
_L = 16
_NW = _NC * _NS


def _make_gather(N, E, D, C, K, dtype=jnp.float32):
    per_w = E // _NW
    G = K * C
    n_groups = per_w // G
    mesh = plsc.VectorSubcoreMesh(core_axis_name="c", subcore_axis_name="s")

    @functools.partial(
        pl.kernel,
        out_type=jax.ShapeDtypeStruct((E, D), dtype),
        mesh=mesh,
        scratch_types=(
            [pltpu.VMEM((G,), jnp.int32) for _ in range(2)]
            + [pltpu.VMEM((G, D), dtype) for _ in range(2)]
            + [pltpu.SemaphoreType.DMA for _ in range(6)]
        ),
    )
    def gather_k(x_hbm, row_hbm, out_hbm, *scr):
        idx_v = scr[:2]
        rows_v = scr[2:4]
        g0, g1, w0, w1, i0, i1 = scr[4:]
        wid = lax.axis_index("s") * _NC + lax.axis_index("c")
        base = wid * per_w
        gsem = (g0, g1)
        wsem = (w0, w1)
        isem = (i0, i1)
        wb = [None, None]
        idx_loads = [None, None]
        gathers = [None, None]

        def load_idx(g):
            b = g % 2
            off = base + g * G
            idx_loads[b] = [
                pltpu.async_copy(row_hbm.at[pl.ds(off, G)], idx_v[b], isem[b])
            ]

        load_idx(0)
        for g in range(n_groups):
            b = g % 2
            off = base + g * G
            for d in idx_loads[b]:
                d.wait()
            if wb[b] is not None:
                wb[b].wait()
            gathers[b] = [
                pltpu.async_copy(
                    x_hbm.at[idx_v[b].at[pl.ds(j * C, C)]],
                    rows_v[b].at[pl.ds(j * C, C)], gsem[b])
                for j in range(K)
            ]
            pb = 1 - b
            if gathers[pb] is not None:
                for d in gathers[pb]:
                    d.wait()
                gathers[pb] = None
                wb[pb] = pltpu.async_copy(
                    rows_v[pb], out_hbm.at[pl.ds(base + (g - 1) * G, G)],
                    wsem[pb])
            if g + 1 < n_groups:
                load_idx(g + 1)
        lb = (n_groups - 1) % 2
        for d in gathers[lb]:
            d.wait()
        wb[lb] = pltpu.async_copy(
            rows_v[lb], out_hbm.at[pl.ds(base + (n_groups - 1) * G, G)],
            wsem[lb])
        for d in wb:
            if d is not None:
                d.wait()

    return gather_k


def _make_scatter(N_pad, E, D, C, n_in):
    per_w = E // _NW
    G = C
    n_groups = per_w // G
    rows_per_sub = N_pad // _NS
    mesh = plsc.VectorSubcoreMesh(core_axis_name="c", subcore_axis_name="s")

    @functools.partial(
        pl.kernel,
        out_type=(
            jax.ShapeDtypeStruct((_NC, N_pad, D), jnp.float32),
            jax.ShapeDtypeStruct((_NC, N_pad), jnp.float32),
        ),
        mesh=mesh,
        scratch_types=(
            [pltpu.VMEM((C,), jnp.int32) for _ in range(2)]
            + [pltpu.VMEM((G, D), jnp.float32) for _ in range(2)]
            + [
                pltpu.VMEM((C,), jnp.float32),
                pltpu.VMEM_SHARED((N_pad, D), jnp.float32),
                pltpu.VMEM_SHARED((N_pad,), jnp.float32),
            ]
            + [pltpu.SemaphoreType.DMA for _ in range(4)]
        ),
    )
    def scatter_k(*args):
        m_hbms = args[:n_in]
        col_hbms = args[n_in:2 * n_in]
        zeros_hbm, zeros_c_hbm, ones_hbm, sums_hbm, cnt_hbm = \
            args[2 * n_in:2 * n_in + 5]
        scr = args[2 * n_in + 5:]
        idx_v = scr[:2]
        rows_v = scr[2:4]
        ones_v, accum, cnt_accum = scr[4:7]
        lsem = scr[7:9]
        asem = scr[9:11]
        cid = lax.axis_index("c")
        sid = lax.axis_index("s")
        wid = sid * _NC + cid

        pltpu.sync_copy(zeros_hbm.at[pl.ds(sid * rows_per_sub, rows_per_sub)],
                        accum.at[pl.ds(sid * rows_per_sub, rows_per_sub)])
        pltpu.sync_copy(zeros_c_hbm.at[pl.ds(sid * rows_per_sub, rows_per_sub)],
                        cnt_accum.at[pl.ds(sid * rows_per_sub, rows_per_sub)])
        pltpu.sync_copy(ones_hbm, ones_v)
        plsc.subcore_barrier()

        base = wid * per_w
        adds = [None, None]
        loads = [None, None]
        n_tot = n_in * n_groups

        def load(g):
            t, gg = divmod(g, n_groups)
            b = g % 2
            off = base + gg * G
            loads[b] = [
                pltpu.async_copy(col_hbms[t].at[pl.ds(off, C)], idx_v[b],
                                 lsem[b]),
                pltpu.async_copy(m_hbms[t].at[pl.ds(off, G)], rows_v[b],
                                 lsem[b]),
            ]

        load(0)
        for g in range(n_tot):
            b = g % 2
            for d in loads[b]:
                d.wait()
            adds[b] = [
                pltpu.async_copy(rows_v[b], accum.at[idx_v[b]],
                                 asem[b], add=True),
                pltpu.async_copy(ones_v, cnt_accum.at[idx_v[b]],
                                 asem[b], add=True),
            ]
            pb = 1 - b
            if adds[pb] is not None:
                for d in adds[pb]:
                    d.wait()
                adds[pb] = None
            if g + 1 < n_tot:
                load(g + 1)
        for fl in adds:
            if fl is not None:
                for d in fl:
                    d.wait()
        plsc.subcore_barrier()

        pltpu.sync_copy(accum.at[pl.ds(sid * rows_per_sub, rows_per_sub)],
                        sums_hbm.at[cid, pl.ds(sid * rows_per_sub, rows_per_sub)])
        pltpu.sync_copy(cnt_accum.at[pl.ds(sid * rows_per_sub, rows_per_sub)],
                        cnt_hbm.at[cid, pl.ds(sid * rows_per_sub, rows_per_sub)])

    return scatter_k


def _edge_mlp(xg, ea, w_x, w_e, b1, w2, b2, Be):
    E, D = xg.shape
    Fe = ea.shape[1]
    H1 = w_x.shape[1]
    H2 = w2.shape[1]

    def body(xg_ref, ea_ref, wx_ref, we_ref, b1_ref, w2_ref, b2_ref, out_ref):
        xb = xg_ref[...].astype(wx_ref.dtype)
        h = jnp.dot(xb, wx_ref[...], preferred_element_type=jnp.float32)
        h = h + jnp.dot(ea_ref[...], we_ref[...], preferred_element_type=jnp.float32)
        h = jnp.maximum(h + b1_ref[...], 0.0).astype(w2_ref.dtype)
        out_ref[...] = (
            jnp.dot(h, w2_ref[...], preferred_element_type=jnp.float32) + b2_ref[...]
        )

    return pl.pallas_call(
        body,
        grid=(E // Be,),
        in_specs=[
            pl.BlockSpec((Be, D), lambda i: (i, 0)),
            pl.BlockSpec((Be, Fe), lambda i: (i, 0)),
            pl.BlockSpec((D, H1), lambda i: (0, 0)),
            pl.BlockSpec((Fe, H1), lambda i: (0, 0)),
            pl.BlockSpec((1, H1), lambda i: (0, 0)),
            pl.BlockSpec((H1, H2), lambda i: (0, 0)),
            pl.BlockSpec((1, H2), lambda i: (0, 0)),
        ],
        out_specs=pl.BlockSpec((Be, H2), lambda i: (i, 0)),
        out_shape=jax.ShapeDtypeStruct((E, H2), jnp.float32),
    )(xg, ea, w_x, w_e, b1, w2, b2)


def _node_mlp(x_p, sums_list, cnt_list, batch3, u,
              w_x, w_m, w_u, b_a, w_b, b_b, Bn):
    N_pad, D = x_p.shape
    Bsz, Fu = u.shape
    H = w_x.shape[1]
    H3 = w_b.shape[1]
    nblk = N_pad // Bn
    T = len(sums_list)

    def body(*refs):
        x_ref = refs[0]
        s_refs = refs[1:1 + T]
        c_refs = refs[1 + T:1 + 2 * T]
        b_ref, u_ref, wx_ref, wm_ref, wu_ref, ba_ref, wb_ref, bb_ref, out_ref = \
            refs[1 + 2 * T:]
        c = c_refs[0][0] + c_refs[0][1]
        s = s_refs[0][0] + s_refs[0][1]
        for t in range(1, T):
            c = c + c_refs[t][0] + c_refs[t][1]
            s = s + s_refs[t][0] + s_refs[t][1]
        mean = s / jnp.maximum(c, 1.0)[:, None]
        bidx = b_ref[0, 0]
        oh = (bidx[:, None] == lax.broadcasted_iota(jnp.int32, (Bn, Bsz), 1)
              ).astype(jnp.float32)
        ub = jnp.dot(oh, u_ref[...], preferred_element_type=jnp.float32)
        h = jnp.dot(x_ref[...], wx_ref[...], preferred_element_type=jnp.float32)
        h = h + jnp.dot(mean, wm_ref[...], preferred_element_type=jnp.float32)
        h = h + jnp.dot(ub, wu_ref[...], preferred_element_type=jnp.float32)
        h = jnp.maximum(h + ba_ref[...], 0.0)
        out_ref[...] = (
            jnp.dot(h, wb_ref[...], preferred_element_type=jnp.float32) + bb_ref[...]
        )

    return pl.pallas_call(
        body,
        grid=(nblk,),
        in_specs=(
            [pl.BlockSpec((Bn, D), lambda i: (i, 0))]
            + [pl.BlockSpec((_NC, Bn, D), lambda i: (0, i, 0))] * T
            + [pl.BlockSpec((_NC, Bn), lambda i: (0, i))] * T
            + [
                pl.BlockSpec((1, 1, Bn), lambda i: (i, 0, 0)),
                pl.BlockSpec((Bsz, Fu), lambda i: (0, 0)),
                pl.BlockSpec((D, H), lambda i: (0, 0)),
                pl.BlockSpec((D, H), lambda i: (0, 0)),
                pl.BlockSpec((Fu, H), lambda i: (0, 0)),
                pl.BlockSpec((1, H), lambda i: (0, 0)),
                pl.BlockSpec((H, H3), lambda i: (0, 0)),
                pl.BlockSpec((1, H3), lambda i: (0, 0)),
            ]
        ),
        out_specs=pl.BlockSpec((Bn, H3), lambda i: (i, 0)),
        out_shape=jax.ShapeDtypeStruct((N_pad, H3), jnp.float32),
    )(x_p, *sums_list, *cnt_list, batch3, u, w_x, w_m, w_u, b_a, w_b, b_b)


def kernel(x, edge_index, edge_attr, u, batch,
           W1a, b1a, W1b, b1b, W2a, b2a, W2b, b2b):
    N, F_x = x.shape
    E = edge_attr.shape[0]
    Bsz, F_u = u.shape
    h2 = W1b.shape[1]

    row = edge_index[0].astype(jnp.int32)
    col = edge_index[1].astype(jnp.int32)

    C = 128
    K = 2
    Bn = 512
    N_pad = 10240
    T = 5
    Ec = 65536
    E_pad = T * Ec
    pad = E_pad - E

    zeros = jnp.zeros((N_pad, h2), jnp.float32)
    zeros_c = jnp.zeros((N_pad,), jnp.float32)
    ones = jnp.ones((C,), jnp.float32)
    bf16 = jnp.bfloat16
    row_p = jnp.pad(row, (0, pad))
    col_p = jnp.pad(col, (0, pad), constant_values=N_pad - 2)
    ea_bf = jnp.pad(edge_attr, ((0, pad), (0, 0))).astype(bf16)
    w1x, w1e = W1a[:F_x].astype(bf16), W1a[F_x:].astype(bf16)
    w1b_bf = W1b.astype(bf16)
    b1r = b1a.reshape(1, -1)
    b1br = b1b.reshape(1, -1)

    gather_k = _make_gather(N, Ec, F_x, C, K)
    scatter_k = _make_scatter(N_pad, Ec, h2, C, 1)

    sums_list, cnt_list = [], []
    for t in range(T):
        sl = slice(t * Ec, (t + 1) * Ec)
        xg = gather_k(x, row_p[sl])
        m = _edge_mlp(xg, ea_bf[sl], w1x, w1e, b1r, w1b_bf, b1br, 2048)
        sums, cnt = scatter_k(m, col_p[sl], zeros, zeros_c, ones)
        sums_list.append(sums)
        cnt_list.append(cnt)

    x_p = jnp.pad(x, ((0, N_pad - N), (0, 0)))
    batch3 = jnp.pad(batch.astype(jnp.int32), (0, N_pad - N)).reshape(
        N_pad // Bn, 1, Bn)
    out_p = _node_mlp(x_p, sums_list, cnt_list, batch3, u,
                      W2a[:F_x], W2a[F_x:F_x + h2], W2a[F_x + h2:],
                      b2a.reshape(1, -1), W2b, b2b.reshape(1, -1), Bn)
    return out_p[:N]


# speedup vs baseline: 1.4835x; 1.4835x over previous
import functools

import jax
import jax.numpy as jnp
from jax import lax
from jax.experimental import pallas as pl
from jax.experimental.pallas import tpu as pltpu
from jax.experimental.pallas import tpu_sc as plsc

_NC = 2
_NS = 16
_L = 16
_NW = _NC * _NS


def _make_gather(N, E, D, C, K, dtype=jnp.float32):
    per_w = E // _NW
    G = K * C
    n_groups = per_w // G
    mesh = plsc.VectorSubcoreMesh(core_axis_name="c", subcore_axis_name="s")

    @functools.partial(
        pl.kernel,
        out_type=jax.ShapeDtypeStruct((E, D), dtype),
        mesh=mesh,
        scratch_types=(
            [pltpu.VMEM((C,), jnp.int32) for _ in range(2 * K)]
            + [pltpu.VMEM((G, D), dtype) for _ in range(2)]
            + [pltpu.SemaphoreType.DMA for _ in range(6)]
        ),
    )
    def gather_k(x_hbm, row_hbm, out_hbm, *scr):
        idx_v = [scr[:K], scr[K:2 * K]]
        rows_v = scr[2 * K:2 * K + 2]
        g0, g1, w0, w1, i0, i1 = scr[2 * K + 2:]
        wid = lax.axis_index("s") * _NC + lax.axis_index("c")
        base = wid * per_w
        gsem = (g0, g1)
        wsem = (w0, w1)
        isem = (i0, i1)
        wb = [None, None]
        idx_loads = [None, None]
        gathers = [None, None]

        def load_idx(g):
            b = g % 2
            off = base + g * G
            idx_loads[b] = [
                pltpu.async_copy(row_hbm.at[pl.ds(off + j * C, C)],
                                 idx_v[b][j], isem[b])
                for j in range(K)
            ]

        load_idx(0)
        for g in range(n_groups):
            b = g % 2
            off = base + g * G
            for d in idx_loads[b]:
                d.wait()
            if wb[b] is not None:
                wb[b].wait()
            gathers[b] = [
                pltpu.async_copy(
                    x_hbm.at[idx_v[b][j]],
                    rows_v[b].at[pl.ds(j * C, C)], gsem[b])
                for j in range(K)
            ]
            pb = 1 - b
            if gathers[pb] is not None:
                for d in gathers[pb]:
                    d.wait()
                gathers[pb] = None
                wb[pb] = pltpu.async_copy(
                    rows_v[pb], out_hbm.at[pl.ds(base + (g - 1) * G, G)],
                    wsem[pb])
            if g + 1 < n_groups:
                load_idx(g + 1)
        lb = (n_groups - 1) % 2
        for d in gathers[lb]:
            d.wait()
        wb[lb] = pltpu.async_copy(
            rows_v[lb], out_hbm.at[pl.ds(base + (n_groups - 1) * G, G)],
            wsem[lb])
        for d in wb:
            if d is not None:
                d.wait()

    return gather_k


def _make_scatter(N_pad, E, D, C, n_in):
    per_w = E // _NW
    G = C
    n_groups = per_w // G
    rows_per_sub = N_pad // _NS
    mesh = plsc.VectorSubcoreMesh(core_axis_name="c", subcore_axis_name="s")

    @functools.partial(
        pl.kernel,
        out_type=(
            jax.ShapeDtypeStruct((_NC, N_pad, D), jnp.float32),
            jax.ShapeDtypeStruct((_NC, N_pad), jnp.float32),
        ),
        mesh=mesh,
        scratch_types=(
            [pltpu.VMEM((C,), jnp.int32) for _ in range(3)]
            + [pltpu.VMEM((G, D), jnp.float32) for _ in range(3)]
            + [
                pltpu.VMEM((C,), jnp.float32),
                pltpu.VMEM_SHARED((N_pad, D), jnp.float32),
                pltpu.VMEM_SHARED((N_pad,), jnp.float32),
            ]
            + [pltpu.SemaphoreType.DMA for _ in range(6)]
        ),
    )
    def scatter_k(*args):
        m_hbms = args[:n_in]
        col_hbms = args[n_in:2 * n_in]
        zeros_hbm, zeros_c_hbm, ones_hbm, sums_hbm, cnt_hbm = \
            args[2 * n_in:2 * n_in + 5]
        scr = args[2 * n_in + 5:]
        idx_v = scr[:3]
        rows_v = scr[3:6]
        ones_v, accum, cnt_accum = scr[6:9]
        lsem = scr[9:12]
        asem = scr[12:15]
        cid = lax.axis_index("c")
        sid = lax.axis_index("s")
        wid = sid * _NC + cid

        pltpu.sync_copy(zeros_hbm.at[pl.ds(sid * rows_per_sub, rows_per_sub)],
                        accum.at[pl.ds(sid * rows_per_sub, rows_per_sub)])
        pltpu.sync_copy(zeros_c_hbm.at[pl.ds(sid * rows_per_sub, rows_per_sub)],
                        cnt_accum.at[pl.ds(sid * rows_per_sub, rows_per_sub)])
        pltpu.sync_copy(ones_hbm, ones_v)
        plsc.subcore_barrier()

        base = wid * per_w
        adds = [None, None, None]
        loads = [None, None, None]
        n_tot = n_in * n_groups

        def load(g):
            t, gg = divmod(g, n_groups)
            b = g % 3
            off = base + gg * G
            loads[b] = [
                pltpu.async_copy(col_hbms[t].at[pl.ds(off, C)], idx_v[b],
                                 lsem[b]),
                pltpu.async_copy(m_hbms[t].at[pl.ds(off, G)], rows_v[b],
                                 lsem[b]),
            ]

        load(0)
        for g in range(n_tot):
            b = g % 3
            for d in loads[b]:
                d.wait()
            adds[b] = [
                pltpu.async_copy(rows_v[b], accum.at[idx_v[b]],
                                 asem[b], add=True),
                pltpu.async_copy(ones_v, cnt_accum.at[idx_v[b]],
                                 asem[b], add=True),
            ]
            if g + 1 < n_tot:
                nb = (g + 1) % 3
                if adds[nb] is not None:
                    for d in adds[nb]:
                        d.wait()
                    adds[nb] = None
                load(g + 1)
        for fl in adds:
            if fl is not None:
                for d in fl:
                    d.wait()
        plsc.subcore_barrier()

        pltpu.sync_copy(accum.at[pl.ds(sid * rows_per_sub, rows_per_sub)],
                        sums_hbm.at[cid, pl.ds(sid * rows_per_sub, rows_per_sub)])
        pltpu.sync_copy(cnt_accum.at[pl.ds(sid * rows_per_sub, rows_per_sub)],
                        cnt_hbm.at[cid, pl.ds(sid * rows_per_sub, rows_per_sub)])

    return scatter_k


def _edge_mlp(xg, ea, w_x, w_e, b1, w2, b2, Be):
    E, D = xg.shape
    Fe = ea.shape[1]
    H1 = w_x.shape[1]
    H2 = w2.shape[1]

    def body(xg_ref, ea_ref, wx_ref, we_ref, b1_ref, w2_ref, b2_ref, out_ref):
        xb = xg_ref[...].astype(wx_ref.dtype)
        h = jnp.dot(xb, wx_ref[...], preferred_element_type=jnp.float32)
        h = h + jnp.dot(ea_ref[...], we_ref[...], preferred_element_type=jnp.float32)
        h = jnp.maximum(h + b1_ref[...], 0.0).astype(w2_ref.dtype)
        out_ref[...] = (
            jnp.dot(h, w2_ref[...], preferred_element_type=jnp.float32) + b2_ref[...]
        )

    return pl.pallas_call(
        body,
        grid=(E // Be,),
        in_specs=[
            pl.BlockSpec((Be, D), lambda i: (i, 0)),
            pl.BlockSpec((Be, Fe), lambda i: (i, 0)),
            pl.BlockSpec((D, H1), lambda i: (0, 0)),
            pl.BlockSpec((Fe, H1), lambda i: (0, 0)),
            pl.BlockSpec((1, H1), lambda i: (0, 0)),
            pl.BlockSpec((H1, H2), lambda i: (0, 0)),
            pl.BlockSpec((1, H2), lambda i: (0, 0)),
        ],
        out_specs=pl.BlockSpec((Be, H2), lambda i: (i, 0)),
        out_shape=jax.ShapeDtypeStruct((E, H2), jnp.float32),
    )(xg, ea, w_x, w_e, b1, w2, b2)


def _node_mlp(x_p, sums_list, cnt_list, batch3, u,
              w_x, w_m, w_u, b_a, w_b, b_b, Bn):
    N_pad, D = x_p.shape
    Bsz, Fu = u.shape
    H = w_x.shape[1]
    H3 = w_b.shape[1]
    nblk = N_pad // Bn
    T = len(sums_list)

    def body(*refs):
        x_ref = refs[0]
        s_refs = refs[1:1 + T]
        c_refs = refs[1 + T:1 + 2 * T]
        b_ref, u_ref, wx_ref, wm_ref, wu_ref, ba_ref, wb_ref, bb_ref, out_ref = \
            refs[1 + 2 * T:]
        c = c_refs[0][0] + c_refs[0][1]
        s = s_refs[0][0] + s_refs[0][1]
        for t in range(1, T):
            c = c + c_refs[t][0] + c_refs[t][1]
            s = s + s_refs[t][0] + s_refs[t][1]
        mean = s / jnp.maximum(c, 1.0)[:, None]
        bidx = b_ref[0, 0]
        oh = (bidx[:, None] == lax.broadcasted_iota(jnp.int32, (Bn, Bsz), 1)
              ).astype(jnp.float32)
        ub = jnp.dot(oh, u_ref[...], preferred_element_type=jnp.float32)
        h = jnp.dot(x_ref[...], wx_ref[...], preferred_element_type=jnp.float32)
        h = h + jnp.dot(mean, wm_ref[...], preferred_element_type=jnp.float32)
        h = h + jnp.dot(ub, wu_ref[...], preferred_element_type=jnp.float32)
        h = jnp.maximum(h + ba_ref[...], 0.0)
        out_ref[...] = (
            jnp.dot(h, wb_ref[...], preferred_element_type=jnp.float32) + bb_ref[...]
        )

    return pl.pallas_call(
        body,
        grid=(nblk,),
        in_specs=(
            [pl.BlockSpec((Bn, D), lambda i: (i, 0))]
            + [pl.BlockSpec((_NC, Bn, D), lambda i: (0, i, 0))] * T
            + [pl.BlockSpec((_NC, Bn), lambda i: (0, i))] * T
            + [
                pl.BlockSpec((1, 1, Bn), lambda i: (i, 0, 0)),
                pl.BlockSpec((Bsz, Fu), lambda i: (0, 0)),
                pl.BlockSpec((D, H), lambda i: (0, 0)),
                pl.BlockSpec((D, H), lambda i: (0, 0)),
                pl.BlockSpec((Fu, H), lambda i: (0, 0)),
                pl.BlockSpec((1, H), lambda i: (0, 0)),
                pl.BlockSpec((H, H3), lambda i: (0, 0)),
                pl.BlockSpec((1, H3), lambda i: (0, 0)),
            ]
        ),
        out_specs=pl.BlockSpec((Bn, H3), lambda i: (i, 0)),
        out_shape=jax.ShapeDtypeStruct((N_pad, H3), jnp.float32),
    )(x_p, *sums_list, *cnt_list, batch3, u, w_x, w_m, w_u, b_a, w_b, b_b)


def kernel(x, edge_index, edge_attr, u, batch,
           W1a, b1a, W1b, b1b, W2a, b2a, W2b, b2b):
    N, F_x = x.shape
    E = edge_attr.shape[0]
    Bsz, F_u = u.shape
    h2 = W1b.shape[1]

    row = edge_index[0].astype(jnp.int32)
    col = edge_index[1].astype(jnp.int32)

    C = 80
    K = 5
    Bn = 512
    N_pad = 10240
    T = 5
    Ec = E // T

    zeros = jnp.zeros((N_pad, h2), jnp.float32)
    zeros_c = jnp.zeros((N_pad,), jnp.float32)
    ones = jnp.ones((C,), jnp.float32)
    bf16 = jnp.bfloat16
    ea_bf = edge_attr.astype(bf16)
    w1x, w1e = W1a[:F_x].astype(bf16), W1a[F_x:].astype(bf16)
    w1b_bf = W1b.astype(bf16)
    b1r = b1a.reshape(1, -1)
    b1br = b1b.reshape(1, -1)

    gather_k = _make_gather(N, Ec, F_x, C, K)
    scatter_k = _make_scatter(N_pad, Ec, h2, C, 1)

    sums_list, cnt_list = [], []
    for t in range(T):
        sl = slice(t * Ec, (t + 1) * Ec)
        xg = gather_k(x, row[sl])
        m = _edge_mlp(xg, ea_bf[sl], w1x, w1e, b1r, w1b_bf, b1br, 1280)
        sums, cnt = scatter_k(m, col[sl], zeros, zeros_c, ones)
        sums_list.append(sums)
        cnt_list.append(cnt)

    x_p = jnp.pad(x, ((0, N_pad - N), (0, 0)))
    batch3 = jnp.pad(batch.astype(jnp.int32), (0, N_pad - N)).reshape(
        N_pad // Bn, 1, Bn)
    out_p = _node_mlp(x_p, sums_list, cnt_list, batch3, u,
                      W2a[:F_x], W2a[F_x:F_x + h2], W2a[F_x + h2:],
                      b2a.reshape(1, -1), W2b, b2b.reshape(1, -1), Bn)
    return out_p[:N]
